# Initial kernel scaffold; baseline (speedup 1.0000x reference)
import jax
import jax.numpy as jnp
from jax.experimental import pallas as pl


def kernel(pred, target):
    raise NotImplementedError("write your pallas kernel here")


# speedup vs baseline: 6.6315x; 6.6315x over previous
import functools

import jax
import jax.numpy as jnp
from jax import lax
from jax.experimental import pallas as pl
from jax.experimental.pallas import tpu as pltpu
from jax.experimental.pallas import tpu_sc as plsc

_B = 8
_N = 2048
_L = 16
_NC = 2
_NS = 16
_NCH = _N // _L
_UNROLL = 16
_RB = 256


def _cost_tc_kernel(p_ref, t_ref, o_ref):
    p = p_ref[0]
    t = t_ref[0]
    px, py, pz = p[:, 0:1], p[:, 1:2], p[:, 2:3]
    tx, ty, tz = t[0:1, :], t[1:2, :], t[2:3, :]
    dx = px - tx
    dy = py - ty
    dz = pz - tz
    d2 = dx * dx + dy * dy + dz * dz
    o_ref[0] = jnp.sqrt(jnp.maximum(d2, 1e-12))


_GATHER_DNUMS = lax.GatherDimensionNumbers(
    offset_dims=(), collapsed_slice_dims=(0,), start_index_map=(0,))


def _shuffle(v, perm):
    return lax.gather(v, perm[:, None], _GATHER_DNUMS, (1,),
                      mode=lax.GatherScatterMode.PROMISE_IN_BOUNDS)


def _allmin(v, lane):
    for k in (1, 2, 4, 8):
        v = jnp.minimum(v, _shuffle(v, lane ^ k))
    return v


def _greedy_sc_body(cost_hbm, out_hbm, row_v, pen_v, tot_v):
    cid = lax.axis_index("c")
    sid = lax.axis_index("s")
    wid = sid * _NC + cid

    @pl.when(wid < _B)
    def _():
        zero = jnp.zeros((_L,), jnp.float32)

        def initc(c, carry):
            pen_v[pl.ds(c * _L, _L)] = zero
            return carry

        lax.fori_loop(0, _NCH, initc, 0, unroll=8)

        lane = lax.iota(jnp.int32, _L)
        big_i = jnp.full((_L,), jnp.int32(2 ** 30))
        inf_f = jnp.full((_L,), jnp.float32(jnp.inf))

        def step(i, total):
            pltpu.sync_copy(cost_hbm.at[wid, i], row_v)

            def chunks(cc, carry):
                minv, mini = carry
                for u in range(_UNROLL):
                    c = cc * _UNROLL + u
                    v = row_v[pl.ds(c * _L, _L)] + pen_v[pl.ds(c * _L, _L)]
                    idx = lane + c * _L
                    upd = v < minv
                    minv = jnp.where(upd, v, minv)
                    mini = jnp.where(upd, idx, mini)
                return minv, mini

            minv, mini = lax.fori_loop(0, _NCH // _UNROLL, chunks,
                                       (inf_f, big_i))
            gmin = _allmin(minv, lane)
            cand = jnp.where(minv == gmin, mini, big_i)
            j = _allmin(cand, lane)
            plsc.store_scatter(pen_v, [j], inf_f, mask=(lane == 0))
            return total + gmin

        total = lax.fori_loop(0, _N, step, jnp.zeros((_L,), jnp.float32))
        tot_v[...] = total
        pltpu.sync_copy(tot_v, out_hbm.at[wid])


def _build_cost(pred, target_t):
    return pl.pallas_call(
        _cost_tc_kernel,
        grid=(_B, _N // _RB),
        in_specs=[
            pl.BlockSpec((1, _RB, 3), lambda b, r: (b, r, 0)),
            pl.BlockSpec((1, 3, _N), lambda b, r: (b, 0, 0)),
        ],
        out_specs=pl.BlockSpec((1, _RB, _N), lambda b, r: (b, r, 0)),
        out_shape=jax.ShapeDtypeStruct((_B, _N, _N), jnp.float32),
    )(pred, target_t)


def _run_greedy(cost):
    mesh = plsc.VectorSubcoreMesh(core_axis_name="c", subcore_axis_name="s",
                                  num_cores=_NC, num_subcores=_NS)
    return pl.kernel(
        _greedy_sc_body,
        out_type=jax.ShapeDtypeStruct((_B, _L), jnp.float32),
        mesh=mesh,
        scratch_types=[
            pltpu.VMEM((_N,), jnp.float32),
            pltpu.VMEM((_N,), jnp.float32),
            pltpu.VMEM((_L,), jnp.float32),
        ],
        compiler_params=pltpu.CompilerParams(needs_layout_passes=False),
    )(cost)


@jax.jit
def kernel(pred, target):
    target_t = jnp.transpose(target, (0, 2, 1))
    cost = _build_cost(pred, target_t)
    totals = _run_greedy(cost)
    return jnp.mean(totals[:, 0] / _N)


# speedup vs baseline: 25.3643x; 3.8248x over previous
import functools

import jax
import jax.numpy as jnp
from jax import lax
from jax.experimental import pallas as pl
from jax.experimental.pallas import tpu as pltpu
from jax.experimental.pallas import tpu_sc as plsc

_B = 8
_N = 2048
_L = 16
_NC = 2
_NS = 16
_NCH = _N // _L
_UNROLL = 16
_RB = 256


def _cost_tc_kernel(p_ref, t_ref, o_ref):
    p = p_ref[0]
    t = t_ref[0]
    px, py, pz = p[:, 0:1], p[:, 1:2], p[:, 2:3]
    tx, ty, tz = t[0:1, :], t[1:2, :], t[2:3, :]
    dx = px - tx
    dy = py - ty
    dz = pz - tz
    d2 = dx * dx + dy * dy + dz * dz
    o_ref[0] = jnp.sqrt(jnp.maximum(d2, 1e-12))


_GATHER_DNUMS = lax.GatherDimensionNumbers(
    offset_dims=(), collapsed_slice_dims=(0,), start_index_map=(0,))


def _shuffle(v, perm):
    return lax.gather(v, perm[:, None], _GATHER_DNUMS, (1,),
                      mode=lax.GatherScatterMode.PROMISE_IN_BOUNDS)


def _allmin(v, lane):
    for k in (1, 2, 4, 8):
        v = jnp.minimum(v, _shuffle(v, lane ^ k))
    return v


_RPB = 16
_NBLK = _N // _RPB


def _greedy_sc_body(cost_hbm, out_hbm, blk0, blk1, pen_v, tot_v, sem0, sem1):
    cid = lax.axis_index("c")
    sid = lax.axis_index("s")
    wid = sid * _NC + cid

    @pl.when(wid < _B)
    def _():
        zero = jnp.zeros((_L,), jnp.float32)

        def initc(c, carry):
            pen_v[pl.ds(c * _L, _L)] = zero
            return carry

        lax.fori_loop(0, _NCH, initc, 0, unroll=8)

        lane = lax.iota(jnp.int32, _L)
        big_i = jnp.full((_L,), jnp.int32(2 ** 30))
        inf_f = jnp.full((_L,), jnp.float32(jnp.inf))

        def do_row(blk, r, total):
            def chunks(cc, carry):
                minv, mini = carry
                for u in range(_UNROLL):
                    c = cc * _UNROLL + u
                    v = blk[r, pl.ds(c * _L, _L)] + pen_v[pl.ds(c * _L, _L)]
                    idx = lane + c * _L
                    upd = v < minv
                    minv = jnp.where(upd, v, minv)
                    mini = jnp.where(upd, idx, mini)
                return minv, mini

            minv, mini = lax.fori_loop(0, _NCH // _UNROLL, chunks,
                                       (inf_f, big_i))
            gmin = _allmin(minv, lane)
            cand = jnp.where(minv == gmin, mini, big_i)
            j = _allmin(cand, lane)
            plsc.store_scatter(pen_v, [j], inf_f, mask=(lane == 0))
            return total + gmin

        def fetch(bidx, blk, sem):
            pltpu.make_async_copy(
                cost_hbm.at[wid, pl.ds(bidx * _RPB, _RPB)], blk, sem).start()

        def drain(blk, sem):
            pltpu.make_async_copy(
                cost_hbm.at[wid, pl.ds(0, _RPB)], blk, sem).wait()

        fetch(0, blk0, sem0)
        fetch(1, blk1, sem1)

        def pair(pb, total):
            drain(blk0, sem0)
            total = lax.fori_loop(
                0, _RPB, lambda r, t: do_row(blk0, r, t), total)

            @pl.when(2 * pb + 2 < _NBLK)
            def _():
                fetch(2 * pb + 2, blk0, sem0)

            drain(blk1, sem1)
            total = lax.fori_loop(
                0, _RPB, lambda r, t: do_row(blk1, r, t), total)

            @pl.when(2 * pb + 3 < _NBLK)
            def _():
                fetch(2 * pb + 3, blk1, sem1)

            return total

        total = lax.fori_loop(0, _NBLK // 2, pair,
                              jnp.zeros((_L,), jnp.float32))
        tot_v[...] = total
        pltpu.sync_copy(tot_v, out_hbm.at[wid])


def _build_cost(pred, target_t):
    return pl.pallas_call(
        _cost_tc_kernel,
        grid=(_B, _N // _RB),
        in_specs=[
            pl.BlockSpec((1, _RB, 3), lambda b, r: (b, r, 0)),
            pl.BlockSpec((1, 3, _N), lambda b, r: (b, 0, 0)),
        ],
        out_specs=pl.BlockSpec((1, _RB, _N), lambda b, r: (b, r, 0)),
        out_shape=jax.ShapeDtypeStruct((_B, _N, _N), jnp.float32),
    )(pred, target_t)


def _run_greedy(cost):
    mesh = plsc.VectorSubcoreMesh(core_axis_name="c", subcore_axis_name="s",
                                  num_cores=_NC, num_subcores=_NS)
    return pl.kernel(
        _greedy_sc_body,
        out_type=jax.ShapeDtypeStruct((_B, _L), jnp.float32),
        mesh=mesh,
        scratch_types=[
            pltpu.VMEM((_RPB, _N), jnp.float32),
            pltpu.VMEM((_RPB, _N), jnp.float32),
            pltpu.VMEM((_N,), jnp.float32),
            pltpu.VMEM((_L,), jnp.float32),
            pltpu.SemaphoreType.DMA,
            pltpu.SemaphoreType.DMA,
        ],
        compiler_params=pltpu.CompilerParams(needs_layout_passes=False),
    )(cost)


@jax.jit
def kernel(pred, target):
    target_t = jnp.transpose(target, (0, 2, 1))
    cost = _build_cost(pred, target_t)
    totals = _run_greedy(cost)
    return jnp.mean(totals[:, 0] / _N)
